# Initial kernel scaffold; baseline (speedup 1.0000x reference)
import jax
import jax.numpy as jnp
from jax.experimental import pallas as pl


def kernel(x, pos_weight):
    raise NotImplementedError("write your pallas kernel here")


# speedup vs baseline: 5.0421x; 5.0421x over previous
import jax
import jax.numpy as jnp
from jax.experimental import pallas as pl


def _bcast_body(w_ref, o_ref):
    o_ref[...] = jnp.broadcast_to(w_ref[...][None], o_ref.shape)


def kernel(x, pos_weight):
    batch, seq_len = x.shape
    embed_dim = pos_weight.shape[1]

    bm = 512
    assert seq_len % bm == 0
    grid = (seq_len // bm,)

    out = pl.pallas_call(
        _bcast_body,
        grid=grid,
        in_specs=[pl.BlockSpec((bm, embed_dim), lambda i: (i, 0))],
        out_specs=pl.BlockSpec((batch, bm, embed_dim), lambda i: (0, i, 0)),
        out_shape=jax.ShapeDtypeStruct((batch, seq_len, embed_dim), pos_weight.dtype),
    )(pos_weight)
    return out


# speedup vs baseline: 5.1909x; 1.0295x over previous
import jax
import jax.numpy as jnp
from jax.experimental import pallas as pl


def _bcast_body(w_ref, o_ref):
    o_ref[...] = jnp.broadcast_to(w_ref[...][None], o_ref.shape)


def kernel(x, pos_weight):
    batch, seq_len = x.shape
    embed_dim = pos_weight.shape[1]

    bm = 1024
    assert seq_len % bm == 0
    grid = (seq_len // bm,)

    out = pl.pallas_call(
        _bcast_body,
        grid=grid,
        in_specs=[pl.BlockSpec((bm, embed_dim), lambda i: (i, 0))],
        out_specs=pl.BlockSpec((batch, bm, embed_dim), lambda i: (0, i, 0)),
        out_shape=jax.ShapeDtypeStruct((batch, seq_len, embed_dim), pos_weight.dtype),
    )(pos_weight)
    return out
